# Initial kernel scaffold; baseline (speedup 1.0000x reference)
import functools

import jax
import jax.numpy as jnp
from jax import lax
from jax.experimental import pallas as pl
from jax.experimental.pallas import tpu as pltpu
from jax.experimental.pallas import tpu_sc as plsc

N = 10000
E = 320000
D = 128

NC = 2
NS = 16
NW = NC * NS
CH = 128
NCH = E // CH
CH_PER_W = (NCH + NW - 1) // NW

N_PAD = 10240
SLAB = N_PAD // NS
ZROWS = 40

_mesh = plsc.VectorSubcoreMesh(core_axis_name="c", subcore_axis_name="s")


def _zero_vec16(ref, nwords):
    def body(i, _):
        ref[pl.ds(i * 16, 16)] = jnp.zeros((16,), jnp.float32)
        return 0
    lax.fori_loop(0, nwords // 16, body, 0)


@functools.partial(
    pl.kernel,
    out_type=jax.ShapeDtypeStruct((NC, 2, N_PAD), jnp.float32),
    mesh=_mesh,
    scratch_types=[
        pltpu.VMEM((CH,), jnp.int32),
        pltpu.VMEM((CH,), jnp.float32),
        pltpu.VMEM((SLAB,), jnp.float32),
        pltpu.VMEM_SHARED((N_PAD,), jnp.float32),
        pltpu.VMEM_SHARED((N_PAD,), jnp.float32),
    ],
)
def _sc_degrees(src_hbm, dst_hbm, out_hbm, idx_v, ones_v, zbuf, dsrc_sh, ddst_sh):
    cid = lax.axis_index("c")
    sid = lax.axis_index("s")
    wid = sid * NC + cid

    def fill_ones(i, _):
        ones_v[pl.ds(i * 16, 16)] = jnp.ones((16,), jnp.float32)
        return 0
    lax.fori_loop(0, CH // 16, fill_ones, 0)
    _zero_vec16(zbuf, SLAB)

    pltpu.sync_copy(zbuf, dsrc_sh.at[pl.ds(sid * SLAB, SLAB)])
    pltpu.sync_copy(zbuf, ddst_sh.at[pl.ds(sid * SLAB, SLAB)])
    plsc.subcore_barrier()

    def chunk_body(j, _):
        ch = wid + j * NW

        @pl.when(ch < NCH)
        def _():
            pltpu.sync_copy(src_hbm.at[pl.ds(ch * CH, CH)], idx_v)
            pltpu.sync_copy(ones_v, dsrc_sh.at[idx_v], add=True)
            pltpu.sync_copy(dst_hbm.at[pl.ds(ch * CH, CH)], idx_v)
            pltpu.sync_copy(ones_v, ddst_sh.at[idx_v], add=True)
        return 0

    lax.fori_loop(0, CH_PER_W, chunk_body, 0)
    plsc.subcore_barrier()

    pltpu.sync_copy(dsrc_sh.at[pl.ds(sid * SLAB, SLAB)],
                    out_hbm.at[cid, 0, pl.ds(sid * SLAB, SLAB)])
    pltpu.sync_copy(ddst_sh.at[pl.ds(sid * SLAB, SLAB)],
                    out_hbm.at[cid, 1, pl.ds(sid * SLAB, SLAB)])


@functools.partial(
    pl.kernel,
    out_type=jax.ShapeDtypeStruct((NC, N_PAD, D), jnp.float32),
    mesh=_mesh,
    scratch_types=[
        pltpu.VMEM((CH,), jnp.int32),
        pltpu.VMEM((CH,), jnp.int32),
        pltpu.VMEM((CH, D), jnp.float32),
        pltpu.VMEM((ZROWS, D), jnp.float32),
        pltpu.VMEM_SHARED((N_PAD, D), jnp.float32),
        pltpu.SemaphoreType.DMA,
    ],
)
def _sc_spmm(y_hbm, src_hbm, dst_hbm, out_hbm, srcv, dstv, rows, zbuf, acc, sem):
    cid = lax.axis_index("c")
    sid = lax.axis_index("s")
    wid = sid * NC + cid

    def zrow(i, _):
        def zcol(j, _):
            zbuf[i, pl.ds(j * 16, 16)] = jnp.zeros((16,), jnp.float32)
            return 0
        lax.fori_loop(0, D // 16, zcol, 0)
        return 0
    lax.fori_loop(0, ZROWS, zrow, 0)

    def zslab(t, _):
        pltpu.sync_copy(zbuf, acc.at[pl.ds(sid * SLAB + t * ZROWS, ZROWS)])
        return 0
    lax.fori_loop(0, SLAB // ZROWS, zslab, 0)
    plsc.subcore_barrier()

    def chunk_body(j, _):
        ch = wid + j * NW

        @pl.when(ch < NCH)
        def _():
            pltpu.sync_copy(src_hbm.at[pl.ds(ch * CH, CH)], srcv)
            pltpu.async_copy(y_hbm.at[srcv], rows, sem).wait()
            pltpu.sync_copy(dst_hbm.at[pl.ds(ch * CH, CH)], dstv)
            pltpu.sync_copy(rows, acc.at[dstv], add=True)
        return 0

    lax.fori_loop(0, CH_PER_W, chunk_body, 0)
    plsc.subcore_barrier()

    pltpu.sync_copy(acc.at[pl.ds(sid * SLAB, SLAB)],
                    out_hbm.at[cid, pl.ds(sid * SLAB, SLAB)])


_RB = 1000
_GRID = N // _RB


def _tc_scale_matmul_body(x_ref, n_ref, w_ref, o_ref):
    o_ref[...] = jnp.dot(x_ref[...] * n_ref[...], w_ref[...],
                         preferred_element_type=jnp.float32)


def _tc_scale_matmul(x, norm_out, w):
    return pl.pallas_call(
        _tc_scale_matmul_body,
        grid=(_GRID,),
        in_specs=[
            pl.BlockSpec((_RB, D), lambda i: (i, 0)),
            pl.BlockSpec((_RB, 1), lambda i: (i, 0)),
            pl.BlockSpec((D, D), lambda i: (0, 0)),
        ],
        out_specs=pl.BlockSpec((_RB, D), lambda i: (i, 0)),
        out_shape=jax.ShapeDtypeStruct((N, D), jnp.float32),
    )(x, norm_out, w)


def _tc_mid_body(p0_ref, p1_ref, ni_ref, no_ref, b_ref, w_ref, o_ref):
    h = jnp.maximum((p0_ref[...] + p1_ref[...]) * ni_ref[...] + b_ref[...], 0.0)
    o_ref[...] = jnp.dot(h * no_ref[...], w_ref[...],
                         preferred_element_type=jnp.float32)


def _tc_mid(p0, p1, norm_in, norm_out, b, w):
    return pl.pallas_call(
        _tc_mid_body,
        grid=(_GRID,),
        in_specs=[
            pl.BlockSpec((_RB, D), lambda i: (i, 0)),
            pl.BlockSpec((_RB, D), lambda i: (i, 0)),
            pl.BlockSpec((_RB, 1), lambda i: (i, 0)),
            pl.BlockSpec((_RB, 1), lambda i: (i, 0)),
            pl.BlockSpec((1, D), lambda i: (0, 0)),
            pl.BlockSpec((D, D), lambda i: (0, 0)),
        ],
        out_specs=pl.BlockSpec((_RB, D), lambda i: (i, 0)),
        out_shape=jax.ShapeDtypeStruct((N, D), jnp.float32),
    )(p0, p1, norm_in, norm_out, b, w)


def _tc_head_body(p0_ref, p1_ref, ni_ref, b_ref, w3_ref, b3_ref, o_ref):
    i = pl.program_id(0)

    @pl.when(i == 0)
    def _():
        o_ref[...] = b3_ref[...]

    h = jnp.maximum((p0_ref[...] + p1_ref[...]) * ni_ref[...] + b_ref[...], 0.0)
    o_ref[...] += jnp.sum(jnp.dot(h, w3_ref[...],
                                  preferred_element_type=jnp.float32),
                          axis=0, keepdims=True) * (1.0 / N)


def _tc_head(p0, p1, norm_in, b, w3, b3):
    return pl.pallas_call(
        _tc_head_body,
        grid=(_GRID,),
        in_specs=[
            pl.BlockSpec((_RB, D), lambda i: (i, 0)),
            pl.BlockSpec((_RB, D), lambda i: (i, 0)),
            pl.BlockSpec((_RB, 1), lambda i: (i, 0)),
            pl.BlockSpec((1, D), lambda i: (0, 0)),
            pl.BlockSpec((D, 1), lambda i: (0, 0)),
            pl.BlockSpec((1, 1), lambda i: (0, 0)),
        ],
        out_specs=pl.BlockSpec((1, 1), lambda i: (0, 0)),
        out_shape=jax.ShapeDtypeStruct((1, 1), jnp.float32),
    )(p0, p1, norm_in, b, w3, b3)


def _norm(deg):
    return jnp.where(deg > 0, lax.rsqrt(jnp.maximum(deg, 1.0)), 0.0)


def kernel(x, edge_index, W1, b1, W2, b2, W3, b3):
    src = edge_index[0]
    dst = edge_index[1]

    degp = _sc_degrees(src, dst)
    deg_out = degp[0, 0, :N] + degp[1, 0, :N]
    deg_in = degp[0, 1, :N] + degp[1, 1, :N]
    norm_out = _norm(deg_out).reshape(N, 1)
    norm_in = _norm(deg_in).reshape(N, 1)

    b1r = b1.reshape(1, D)
    b2r = b2.reshape(1, D)
    b3r = b3.reshape(1, 1)

    y1 = _tc_scale_matmul(x, norm_out, W1)
    s1 = _sc_spmm(y1, src, dst)
    y2 = _tc_mid(s1[0, :N], s1[1, :N], norm_in, norm_out, b1r, W2)
    s2 = _sc_spmm(y2, src, dst)
    return _tc_head(s2[0, :N], s2[1, :N], norm_in, b2r, W3, b3r)


# speedup vs baseline: 6.0141x; 6.0141x over previous
import functools

import jax
import jax.numpy as jnp
from jax import lax
from jax.experimental import pallas as pl
from jax.experimental.pallas import tpu as pltpu
from jax.experimental.pallas import tpu_sc as plsc

N = 10000
E = 320000
D = 128

NC = 2
NS = 16
NW = NC * NS
CH = 128
NCH = E // CH
CH_PER_W = (NCH + NW - 1) // NW

N_PAD = 10240
SLAB = N_PAD // NS
ZROWS = 40

def _zero_vec16(ref, nwords):
    def body(i, _):
        ref[pl.ds(i * 16, 16)] = jnp.zeros((16,), jnp.float32)
        return 0
    lax.fori_loop(0, nwords // 16, body, 0)


def _sc_degrees_body(src_hbm, dst_hbm, out_hbm, idx_v, ones_v, zbuf, dsrc_sh, ddst_sh):
    cid = lax.axis_index("c")
    sid = lax.axis_index("s")
    wid = sid * NC + cid

    def fill_ones(i, _):
        ones_v[pl.ds(i * 16, 16)] = jnp.ones((16,), jnp.float32)
        return 0
    lax.fori_loop(0, CH // 16, fill_ones, 0)
    _zero_vec16(zbuf, SLAB)

    pltpu.sync_copy(zbuf, dsrc_sh.at[pl.ds(sid * SLAB, SLAB)])
    pltpu.sync_copy(zbuf, ddst_sh.at[pl.ds(sid * SLAB, SLAB)])
    plsc.subcore_barrier()

    def chunk_body(j, _):
        ch = wid + j * NW

        @pl.when(ch < NCH)
        def _():
            pltpu.sync_copy(src_hbm.at[pl.ds(ch * CH, CH)], idx_v)
            pltpu.sync_copy(ones_v, dsrc_sh.at[idx_v], add=True)
            pltpu.sync_copy(dst_hbm.at[pl.ds(ch * CH, CH)], idx_v)
            pltpu.sync_copy(ones_v, ddst_sh.at[idx_v], add=True)
        return 0

    lax.fori_loop(0, CH_PER_W, chunk_body, 0)
    plsc.subcore_barrier()

    pltpu.sync_copy(dsrc_sh.at[pl.ds(sid * SLAB, SLAB)],
                    out_hbm.at[cid, 0, pl.ds(sid * SLAB, SLAB)])
    pltpu.sync_copy(ddst_sh.at[pl.ds(sid * SLAB, SLAB)],
                    out_hbm.at[cid, 1, pl.ds(sid * SLAB, SLAB)])


def _sc_spmm_body(y_hbm, src_hbm, dst_hbm, out_hbm, srcv, dstv, rows, zbuf, acc, sem):
    cid = lax.axis_index("c")
    sid = lax.axis_index("s")
    wid = sid * NC + cid

    def zrow(i, _):
        def zcol(j, _):
            zbuf[i, pl.ds(j * 16, 16)] = jnp.zeros((16,), jnp.float32)
            return 0
        lax.fori_loop(0, D // 16, zcol, 0)
        return 0
    lax.fori_loop(0, ZROWS, zrow, 0)

    def zslab(t, _):
        pltpu.sync_copy(zbuf, acc.at[pl.ds(sid * SLAB + t * ZROWS, ZROWS)])
        return 0
    lax.fori_loop(0, SLAB // ZROWS, zslab, 0)
    plsc.subcore_barrier()

    def chunk_body(j, _):
        ch = wid + j * NW

        @pl.when(ch < NCH)
        def _():
            pltpu.sync_copy(src_hbm.at[pl.ds(ch * CH, CH)], srcv)
            pltpu.async_copy(y_hbm.at[srcv], rows, sem).wait()
            pltpu.sync_copy(dst_hbm.at[pl.ds(ch * CH, CH)], dstv)
            pltpu.sync_copy(rows, acc.at[dstv], add=True)
        return 0

    lax.fori_loop(0, CH_PER_W, chunk_body, 0)
    plsc.subcore_barrier()

    pltpu.sync_copy(acc.at[pl.ds(sid * SLAB, SLAB)],
                    out_hbm.at[cid, pl.ds(sid * SLAB, SLAB)])


@functools.cache
def _sc_kernels():
    mesh = plsc.VectorSubcoreMesh(core_axis_name="c", subcore_axis_name="s")
    degrees = pl.kernel(
        _sc_degrees_body,
        out_type=jax.ShapeDtypeStruct((NC, 2, N_PAD), jnp.float32),
        mesh=mesh,
        scratch_types=[
            pltpu.VMEM((CH,), jnp.int32),
            pltpu.VMEM((CH,), jnp.float32),
            pltpu.VMEM((SLAB,), jnp.float32),
            pltpu.VMEM_SHARED((N_PAD,), jnp.float32),
            pltpu.VMEM_SHARED((N_PAD,), jnp.float32),
        ],
    )
    spmm = pl.kernel(
        _sc_spmm_body,
        out_type=jax.ShapeDtypeStruct((NC, N_PAD, D), jnp.float32),
        mesh=mesh,
        scratch_types=[
            pltpu.VMEM((CH,), jnp.int32),
            pltpu.VMEM((CH,), jnp.int32),
            pltpu.VMEM((CH, D), jnp.float32),
            pltpu.VMEM((ZROWS, D), jnp.float32),
            pltpu.VMEM_SHARED((N_PAD, D), jnp.float32),
            pltpu.SemaphoreType.DMA,
        ],
    )
    return degrees, spmm


_RB = 1000
_GRID = N // _RB


def _tc_scale_matmul_body(x_ref, n_ref, w_ref, o_ref):
    o_ref[...] = jnp.dot(x_ref[...] * n_ref[...], w_ref[...],
                         preferred_element_type=jnp.float32)


def _tc_scale_matmul(x, norm_out, w):
    return pl.pallas_call(
        _tc_scale_matmul_body,
        grid=(_GRID,),
        in_specs=[
            pl.BlockSpec((_RB, D), lambda i: (i, 0)),
            pl.BlockSpec((_RB, 1), lambda i: (i, 0)),
            pl.BlockSpec((D, D), lambda i: (0, 0)),
        ],
        out_specs=pl.BlockSpec((_RB, D), lambda i: (i, 0)),
        out_shape=jax.ShapeDtypeStruct((N, D), jnp.float32),
    )(x, norm_out, w)


def _tc_mid_body(p0_ref, p1_ref, ni_ref, no_ref, b_ref, w_ref, o_ref):
    h = jnp.maximum((p0_ref[...] + p1_ref[...]) * ni_ref[...] + b_ref[...], 0.0)
    o_ref[...] = jnp.dot(h * no_ref[...], w_ref[...],
                         preferred_element_type=jnp.float32)


def _tc_mid(p0, p1, norm_in, norm_out, b, w):
    return pl.pallas_call(
        _tc_mid_body,
        grid=(_GRID,),
        in_specs=[
            pl.BlockSpec((_RB, D), lambda i: (i, 0)),
            pl.BlockSpec((_RB, D), lambda i: (i, 0)),
            pl.BlockSpec((_RB, 1), lambda i: (i, 0)),
            pl.BlockSpec((_RB, 1), lambda i: (i, 0)),
            pl.BlockSpec((1, D), lambda i: (0, 0)),
            pl.BlockSpec((D, D), lambda i: (0, 0)),
        ],
        out_specs=pl.BlockSpec((_RB, D), lambda i: (i, 0)),
        out_shape=jax.ShapeDtypeStruct((N, D), jnp.float32),
    )(p0, p1, norm_in, norm_out, b, w)


def _tc_head_body(p0_ref, p1_ref, ni_ref, b_ref, w3_ref, b3_ref, o_ref):
    i = pl.program_id(0)

    @pl.when(i == 0)
    def _():
        o_ref[...] = b3_ref[...]

    h = jnp.maximum((p0_ref[...] + p1_ref[...]) * ni_ref[...] + b_ref[...], 0.0)
    o_ref[...] += jnp.sum(jnp.dot(h, w3_ref[...],
                                  preferred_element_type=jnp.float32),
                          axis=0, keepdims=True) * (1.0 / N)


def _tc_head(p0, p1, norm_in, b, w3, b3):
    return pl.pallas_call(
        _tc_head_body,
        grid=(_GRID,),
        in_specs=[
            pl.BlockSpec((_RB, D), lambda i: (i, 0)),
            pl.BlockSpec((_RB, D), lambda i: (i, 0)),
            pl.BlockSpec((_RB, 1), lambda i: (i, 0)),
            pl.BlockSpec((1, D), lambda i: (0, 0)),
            pl.BlockSpec((D, 1), lambda i: (0, 0)),
            pl.BlockSpec((1, 1), lambda i: (0, 0)),
        ],
        out_specs=pl.BlockSpec((1, 1), lambda i: (0, 0)),
        out_shape=jax.ShapeDtypeStruct((1, 1), jnp.float32),
    )(p0, p1, norm_in, b, w3, b3)


def _norm(deg):
    return jnp.where(deg > 0, lax.rsqrt(jnp.maximum(deg, 1.0)), 0.0)


def kernel(x, edge_index, W1, b1, W2, b2, W3, b3):
    src = edge_index[0]
    dst = edge_index[1]

    _sc_degrees, _sc_spmm = _sc_kernels()
    degp = _sc_degrees(src, dst)
    deg_out = degp[0, 0, :N] + degp[1, 0, :N]
    deg_in = degp[0, 1, :N] + degp[1, 1, :N]
    norm_out = _norm(deg_out).reshape(N, 1)
    norm_in = _norm(deg_in).reshape(N, 1)

    b1r = b1.reshape(1, D)
    b2r = b2.reshape(1, D)
    b3r = b3.reshape(1, 1)

    y1 = _tc_scale_matmul(x, norm_out, W1)
    s1 = _sc_spmm(y1, src, dst)
    y2 = _tc_mid(s1[0, :N], s1[1, :N], norm_in, norm_out, b1r, W2)
    s2 = _sc_spmm(y2, src, dst)
    return _tc_head(s2[0, :N], s2[1, :N], norm_in, b2r, W3, b3r)
